# baseline (speedup 1.0000x reference)
import functools

import jax
import jax.numpy as jnp
from jax import lax
from jax.experimental import pallas as pl
from jax.experimental.pallas import tpu as pltpu
from jax.experimental.pallas import tpu_sc as plsc

D = 1024
ROWS = 8192
SEQ = 2048
NW = 32
ROWS_PER_W = ROWS // NW
P = 16
NCHUNK = ROWS_PER_W // P
CH = P * D
LANES = 64 * P


def _sc_body(x_hbm, pos_hbm, out_hbm, xbuf, pbuf, sx0, sx1, sp0, sp1, so0, so1):
    wid = lax.axis_index("s") * 2 + lax.axis_index("c")
    row0 = wid * ROWS_PER_W
    pos0 = lax.rem(row0, SEQ)
    sx = [sx0, sx1]
    sp = [sp0, sp1]
    so = [so0, so1]

    def start_in(c, slot):
        off = (row0 + c * P) * D
        poff = (pos0 + c * P) * D
        cx = pltpu.async_copy(x_hbm.at[pl.ds(off, CH)], xbuf.at[slot], sx[slot])
        cp = pltpu.async_copy(pos_hbm.at[pl.ds(poff, CH)], pbuf.at[slot], sp[slot])
        return cx, cp

    def start_out(c, slot):
        off = (row0 + c * P) * D
        return pltpu.async_copy(xbuf.at[slot], out_hbm.at[pl.ds(off, CH)], so[slot])

    pend_in = start_in(0, 0)
    pend_out = [None, None]
    for c in range(NCHUNK):
        slot = c % 2
        nxt = (c + 1) % 2
        if c + 1 < NCHUNK:
            if pend_out[nxt] is not None:
                pend_out[nxt].wait()
                pend_out[nxt] = None
            nxt_in = start_in(c + 1, nxt)
        cx, cp = pend_in
        cx.wait()
        cp.wait()

        def add_body(j, _):
            sl = pl.ds(j * 16, 16)
            xbuf[slot, sl] = xbuf[slot, sl] + pbuf[slot, sl]
            return _

        lax.fori_loop(0, LANES, add_body, 0)
        pend_out[slot] = start_out(c, slot)
        if c + 1 < NCHUNK:
            pend_in = nxt_in
    for p in pend_out:
        if p is not None:
            p.wait()


@jax.jit
def _sc_add(x_flat, pos_flat):
    mesh = plsc.VectorSubcoreMesh(core_axis_name="c", subcore_axis_name="s")
    return pl.kernel(
        _sc_body,
        out_type=jax.ShapeDtypeStruct((ROWS * D,), jnp.float32),
        mesh=mesh,
        scratch_types=[
            pltpu.VMEM((2, CH), jnp.float32),
            pltpu.VMEM((2, CH), jnp.float32),
            pltpu.SemaphoreType.DMA,
            pltpu.SemaphoreType.DMA,
            pltpu.SemaphoreType.DMA,
            pltpu.SemaphoreType.DMA,
            pltpu.SemaphoreType.DMA,
            pltpu.SemaphoreType.DMA,
        ],
    )(x_flat, pos_flat)


def kernel(x, pos_table):
    batch, seq_len, d = x.shape
    out = _sc_add(x.reshape(-1), pos_table.reshape(-1))
    return out.reshape(batch, seq_len, d)


# speedup vs baseline: 1.3755x; 1.3755x over previous
import functools

import jax
import jax.numpy as jnp
from jax import lax
from jax.experimental import pallas as pl
from jax.experimental.pallas import tpu as pltpu
from jax.experimental.pallas import tpu_sc as plsc

D = 1024
ROWS = 8192
SEQ = 2048
NW = 32
ROWS_PER_W = ROWS // NW
P = 16
NCHUNK = ROWS_PER_W // P
CH = P * D
LANES = 64 * P


def _sc_body(x_hbm, pos_hbm, out_hbm, xbuf, pbuf, sx0, sx1, sp0, sp1, so0, so1):
    wid = lax.axis_index("s") * 2 + lax.axis_index("c")
    row0 = wid * ROWS_PER_W
    pos0 = lax.rem(row0, SEQ)
    sx = [sx0, sx1]
    sp = [sp0, sp1]
    so = [so0, so1]

    def start_in(c, slot):
        off = (row0 + c * P) * D
        poff = (pos0 + c * P) * D
        cx = pltpu.async_copy(x_hbm.at[pl.ds(off, CH)], xbuf.at[slot], sx[slot])
        cp = pltpu.async_copy(pos_hbm.at[pl.ds(poff, CH)], pbuf.at[slot], sp[slot])
        return cx, cp

    def start_out(c, slot):
        off = (row0 + c * P) * D
        return pltpu.async_copy(xbuf.at[slot], out_hbm.at[pl.ds(off, CH)], so[slot])

    pend_in = start_in(0, 0)
    pend_out = [None, None]
    for c in range(NCHUNK):
        slot = c % 2
        nxt = (c + 1) % 2
        if c + 1 < NCHUNK:
            if pend_out[nxt] is not None:
                pend_out[nxt].wait()
                pend_out[nxt] = None
            nxt_in = start_in(c + 1, nxt)
        cx, cp = pend_in
        cx.wait()
        cp.wait()

        @plsc.parallel_loop(0, LANES, unroll=8)
        def _(j):
            sl = pl.ds(j * 16, 16)
            xbuf[slot, sl] = xbuf[slot, sl] + pbuf[slot, sl]
        pend_out[slot] = start_out(c, slot)
        if c + 1 < NCHUNK:
            pend_in = nxt_in
    for p in pend_out:
        if p is not None:
            p.wait()


@jax.jit
def _sc_add(x_flat, pos_flat):
    mesh = plsc.VectorSubcoreMesh(core_axis_name="c", subcore_axis_name="s")
    return pl.kernel(
        _sc_body,
        out_type=jax.ShapeDtypeStruct((ROWS * D,), jnp.float32),
        mesh=mesh,
        scratch_types=[
            pltpu.VMEM((2, CH), jnp.float32),
            pltpu.VMEM((2, CH), jnp.float32),
            pltpu.SemaphoreType.DMA,
            pltpu.SemaphoreType.DMA,
            pltpu.SemaphoreType.DMA,
            pltpu.SemaphoreType.DMA,
            pltpu.SemaphoreType.DMA,
            pltpu.SemaphoreType.DMA,
        ],
    )(x_flat, pos_flat)


def kernel(x, pos_table):
    batch, seq_len, d = x.shape
    out = _sc_add(x.reshape(-1), pos_table.reshape(-1))
    return out.reshape(batch, seq_len, d)


# speedup vs baseline: 2.7766x; 2.0186x over previous
import functools

import jax
import jax.numpy as jnp
from jax import lax
from jax.experimental import pallas as pl
from jax.experimental.pallas import tpu as pltpu
from jax.experimental.pallas import tpu_sc as plsc

D = 1024
BATCH = 4
SEQ = 2048
NW = 32
WPB = NW // BATCH
ROWS_PER_W = SEQ // WPB
P = 16
NCHUNK = ROWS_PER_W // P
CH = P * D
LANES = CH // 16


def _sc_body(x_hbm, pos_hbm, out_hbm, xbuf, pbuf, sx0, sx1, sp0, sp1, so0, so1):
    wid = lax.axis_index("s") * 2 + lax.axis_index("c")
    b = wid // WPB
    pos0 = (wid % WPB) * ROWS_PER_W
    sx = [sx0, sx1]
    sp = [sp0, sp1]
    so = [so0, so1]

    def start_in(c, slot):
        r = pos0 + c * P
        cx = pltpu.async_copy(x_hbm.at[b, pl.ds(r, P), :], xbuf.at[slot], sx[slot])
        cp = pltpu.async_copy(pos_hbm.at[pl.ds(r, P), :], pbuf.at[slot], sp[slot])
        return cx, cp

    def start_out(c, slot):
        r = pos0 + c * P
        return pltpu.async_copy(xbuf.at[slot], out_hbm.at[b, pl.ds(r, P), :], so[slot])

    pend_in = start_in(0, 0)
    pend_out = [None, None]
    for c in range(NCHUNK):
        slot = c % 2
        nxt = (c + 1) % 2
        if c + 1 < NCHUNK:
            if pend_out[nxt] is not None:
                pend_out[nxt].wait()
                pend_out[nxt] = None
            nxt_in = start_in(c + 1, nxt)
        cx, cp = pend_in
        cx.wait()
        cp.wait()

        @plsc.parallel_loop(0, P)
        def _(i):
            for j in range(0, D, 16):
                sl = pl.ds(j, 16)
                xbuf[slot, i, sl] = xbuf[slot, i, sl] + pbuf[slot, i, sl]

        pend_out[slot] = start_out(c, slot)
        if c + 1 < NCHUNK:
            pend_in = nxt_in
    for pnd in pend_out:
        if pnd is not None:
            pnd.wait()


@jax.jit
def _sc_add(x, pos_table):
    mesh = plsc.VectorSubcoreMesh(core_axis_name="c", subcore_axis_name="s")
    return pl.kernel(
        _sc_body,
        out_type=jax.ShapeDtypeStruct((BATCH, SEQ, D), jnp.float32),
        mesh=mesh,
        scratch_types=[
            pltpu.VMEM((2, P, D), jnp.float32),
            pltpu.VMEM((2, P, D), jnp.float32),
            pltpu.SemaphoreType.DMA,
            pltpu.SemaphoreType.DMA,
            pltpu.SemaphoreType.DMA,
            pltpu.SemaphoreType.DMA,
            pltpu.SemaphoreType.DMA,
            pltpu.SemaphoreType.DMA,
        ],
        compiler_params=pltpu.CompilerParams(use_tc_tiling_on_sc=True),
    )(x, pos_table)


def kernel(x, pos_table):
    return _sc_add(x, pos_table)


# speedup vs baseline: 3.1188x; 1.1232x over previous
import jax
import jax.numpy as jnp
from jax import lax
from jax.experimental import pallas as pl
from jax.experimental.pallas import tpu as pltpu
from jax.experimental.pallas import tpu_sc as plsc

D = 1024
BATCH = 4
SEQ = 2048
NSC = 2
NTILE = 16
RPT = SEQ // (NSC * NTILE)
P = 16
NSUB = RPT // P
NCH = BATCH * NSUB
NPAIR = NCH // 2


def _sc_body(x_hbm, pos_hbm, out_hbm, pbuf, xbuf, spos, sx0, sx1, so0, so1):
    sc = lax.axis_index("c")
    t = lax.axis_index("s")
    row0 = sc * (NTILE * RPT) + t * RPT
    sx = [sx0, sx1]
    so = [so0, so1]

    def start_in(c, slot):
        b = c // NSUB
        r = row0 + (c % NSUB) * P
        return pltpu.async_copy(x_hbm.at[b, pl.ds(r, P), :], xbuf.at[slot], sx[slot])

    def start_out(c, slot):
        b = c // NSUB
        r = row0 + (c % NSUB) * P
        return pltpu.async_copy(xbuf.at[slot], out_hbm.at[b, pl.ds(r, P), :], so[slot])

    def wait_in(slot):
        pltpu.make_async_copy(x_hbm.at[0, pl.ds(0, P), :], xbuf.at[slot], sx[slot]).wait()

    def wait_out(slot):
        pltpu.make_async_copy(xbuf.at[slot], out_hbm.at[0, pl.ds(0, P), :], so[slot]).wait()

    def add(c, slot):
        sub = c % NSUB

        @plsc.parallel_loop(0, P)
        def _(i):
            prow = sub * P + i
            for j in range(0, D, 16):
                sl = pl.ds(j, 16)
                plsc.addupdate(xbuf.at[slot, i, sl], pbuf[prow, sl])

    cpos = pltpu.async_copy(pos_hbm.at[pl.ds(row0, RPT), :], pbuf, spos)
    start_in(0, 0)
    cpos.wait()

    def pair(cc, carry):
        c0 = 2 * cc
        c1 = c0 + 1

        @pl.when(cc > 0)
        def _():
            wait_out(1)

        start_in(c1, 1)
        wait_in(0)
        add(c0, 0)
        start_out(c0, 0)
        wait_in(1)
        add(c1, 1)

        @pl.when(cc < NPAIR - 1)
        def _():
            wait_out(0)
            start_in(c0 + 2, 0)

        start_out(c1, 1)
        return carry

    lax.fori_loop(0, NPAIR, pair, 0)
    wait_out(0)
    wait_out(1)


@jax.jit
def _sc_add(x, pos_table):
    mesh = plsc.VectorSubcoreMesh(core_axis_name="c", subcore_axis_name="s")
    return pl.kernel(
        _sc_body,
        out_type=jax.ShapeDtypeStruct((BATCH, SEQ, D), jnp.float32),
        mesh=mesh,
        scratch_types=[
            pltpu.VMEM((RPT, D), jnp.float32),
            pltpu.VMEM((2, P, D), jnp.float32),
            pltpu.SemaphoreType.DMA,
            pltpu.SemaphoreType.DMA,
            pltpu.SemaphoreType.DMA,
            pltpu.SemaphoreType.DMA,
            pltpu.SemaphoreType.DMA,
        ],
        compiler_params=pltpu.CompilerParams(use_tc_tiling_on_sc=True),
    )(x, pos_table)


def kernel(x, pos_table):
    return _sc_add(x, pos_table)


# speedup vs baseline: 3.6097x; 1.1574x over previous
import jax
import jax.numpy as jnp
from jax import lax
from jax.experimental import pallas as pl
from jax.experimental.pallas import tpu as pltpu
from jax.experimental.pallas import tpu_sc as plsc

D = 1024
BATCH = 4
SEQ = 2048
NSC = 2
NTILE = 16
RPT = SEQ // (NSC * NTILE)
P = 16
NSUB = RPT // P
NCH = BATCH * NSUB
NPAIR = NCH // 2


NSLOT = 3


def _sc_body(x_hbm, pos_hbm, out_hbm, pbuf, xbuf, spos, sin, sout):
    sc = lax.axis_index("c")
    t = lax.axis_index("s")
    row0 = sc * (NTILE * RPT) + t * RPT

    def loc(c):
        return c // NSUB, row0 + lax.rem(c, NSUB) * P, lax.rem(c, NSLOT)

    def start_in(c):
        b, r, slot = loc(c)
        pltpu.async_copy(x_hbm.at[b, pl.ds(r, P), :], xbuf.at[slot], sin.at[slot])

    def start_out(c):
        b, r, slot = loc(c)
        pltpu.async_copy(xbuf.at[slot], out_hbm.at[b, pl.ds(r, P), :], sout.at[slot])

    def wait_in(c):
        slot = lax.rem(c, NSLOT)
        pltpu.make_async_copy(
            x_hbm.at[0, pl.ds(0, P), :], xbuf.at[slot], sin.at[slot]
        ).wait()

    def wait_out(c):
        slot = lax.rem(c, NSLOT)
        pltpu.make_async_copy(
            xbuf.at[slot], out_hbm.at[0, pl.ds(0, P), :], sout.at[slot]
        ).wait()

    def add(c):
        slot = lax.rem(c, NSLOT)
        sub = lax.rem(c, NSUB)

        @plsc.parallel_loop(0, P)
        def _(i):
            prow = sub * P + i
            for j in range(0, D, 16):
                sl = pl.ds(j, 16)
                plsc.addupdate(xbuf.at[slot, i, sl], pbuf[prow, sl])

    cpos = pltpu.async_copy(pos_hbm.at[pl.ds(row0, RPT), :], pbuf, spos)
    start_in(0)
    start_in(1)
    cpos.wait()

    def body(c, carry):
        @pl.when(c + 2 < NCH)
        def _():
            @pl.when(c >= 1)
            def _():
                wait_out(c - 1)

            start_in(c + 2)

        wait_in(c)
        add(c)
        start_out(c)
        return carry

    lax.fori_loop(0, NCH, body, 0)
    wait_out(NCH - 3)
    wait_out(NCH - 2)
    wait_out(NCH - 1)


@jax.jit
def _sc_add(x, pos_table):
    mesh = plsc.VectorSubcoreMesh(core_axis_name="c", subcore_axis_name="s")
    return pl.kernel(
        _sc_body,
        out_type=jax.ShapeDtypeStruct((BATCH, SEQ, D), jnp.float32),
        mesh=mesh,
        scratch_types=[
            pltpu.VMEM((RPT, D), jnp.float32),
            pltpu.VMEM((NSLOT, P, D), jnp.float32),
            pltpu.SemaphoreType.DMA,
            pltpu.SemaphoreType.DMA((NSLOT,)),
            pltpu.SemaphoreType.DMA((NSLOT,)),
        ],
        compiler_params=pltpu.CompilerParams(use_tc_tiling_on_sc=True),
    )(x, pos_table)


def kernel(x, pos_table):
    return _sc_add(x, pos_table)


# speedup vs baseline: 4.1994x; 1.1634x over previous
import jax
import jax.numpy as jnp
from jax import lax
from jax.experimental import pallas as pl
from jax.experimental.pallas import tpu as pltpu
from jax.experimental.pallas import tpu_sc as plsc

D = 1024
BATCH = 4
SEQ = 2048
NSC = 2
NTILE = 16
RPT = SEQ // (NSC * NTILE)
P = 8
NSUB = RPT // P
NCH = BATCH * NSUB


NSLOT = 6
KPRE = NSLOT - 2


def _sc_body(x_hbm, pos_hbm, out_hbm, pbuf, xbuf, spos, sin, sout):
    sc = lax.axis_index("c")
    t = lax.axis_index("s")
    row0 = sc * (NTILE * RPT) + t * RPT

    def loc(c):
        return c // NSUB, row0 + lax.rem(c, NSUB) * P, lax.rem(c, NSLOT)

    def start_in(c):
        b, r, slot = loc(c)
        pltpu.async_copy(x_hbm.at[b, pl.ds(r, P), :], xbuf.at[slot], sin.at[slot])

    def start_out(c):
        b, r, slot = loc(c)
        pltpu.async_copy(xbuf.at[slot], out_hbm.at[b, pl.ds(r, P), :], sout.at[slot])

    def wait_in(c):
        slot = lax.rem(c, NSLOT)
        pltpu.make_async_copy(
            x_hbm.at[0, pl.ds(0, P), :], xbuf.at[slot], sin.at[slot]
        ).wait()

    def wait_out(c):
        slot = lax.rem(c, NSLOT)
        pltpu.make_async_copy(
            xbuf.at[slot], out_hbm.at[0, pl.ds(0, P), :], sout.at[slot]
        ).wait()

    def add(c):
        slot = lax.rem(c, NSLOT)
        sub = lax.rem(c, NSUB)

        @plsc.parallel_loop(0, P)
        def _(i):
            prow = sub * P + i
            for j in range(0, D, 16):
                sl = pl.ds(j, 16)
                plsc.addupdate(xbuf.at[slot, i, sl], pbuf[prow, sl])

    cpos = pltpu.async_copy(pos_hbm.at[pl.ds(row0, RPT), :], pbuf, spos)
    for c0 in range(KPRE):
        start_in(c0)
    cpos.wait()

    def body(c, carry):
        @pl.when(c + KPRE < NCH)
        def _():
            @pl.when(c + KPRE >= NSLOT)
            def _():
                wait_out(c + KPRE - NSLOT)

            start_in(c + KPRE)

        wait_in(c)
        add(c)
        start_out(c)
        return carry

    lax.fori_loop(0, NCH, body, 0)
    for c0 in range(NCH - NSLOT, NCH):
        wait_out(c0)


@jax.jit
def _sc_add(x, pos_table):
    mesh = plsc.VectorSubcoreMesh(core_axis_name="c", subcore_axis_name="s")
    return pl.kernel(
        _sc_body,
        out_type=jax.ShapeDtypeStruct((BATCH, SEQ, D), jnp.float32),
        mesh=mesh,
        scratch_types=[
            pltpu.VMEM((RPT, D), jnp.float32),
            pltpu.VMEM((NSLOT, P, D), jnp.float32),
            pltpu.SemaphoreType.DMA,
            pltpu.SemaphoreType.DMA((NSLOT,)),
            pltpu.SemaphoreType.DMA((NSLOT,)),
        ],
        compiler_params=pltpu.CompilerParams(use_tc_tiling_on_sc=True),
    )(x, pos_table)


def kernel(x, pos_table):
    return _sc_add(x, pos_table)
